# baseline (speedup 1.0000x reference)
import jax
import jax.numpy as jnp
from jax import lax
from jax.experimental import pallas as pl
from jax.experimental.pallas import tpu as pltpu
from jax.experimental.pallas import tpu_sc as plsc

_USER_NUM = 1000000
_ITEM_NUM = 100000
_FACTOR = 32
_BATCH = 16384
_LAMDA = 0.001

_NC = 2
_NS = 16
_NW = _NC * _NS
_ROWS_PER_W = _BATCH // _NW
_CH = 64
_NCHUNK = _ROWS_PER_W // _CH
_NBLK = _CH // 16


def _sc_body(user_idx, item_idx, pos_user_idx, pos_item_idx, neg_user_idx,
             neg_item_idx, rel_idx, neg_rel_idx,
             user_t, item_t, urel_t, irel_t,
             xhat_out, n_u, n_ur, n_i, n_ir, n_pu, n_pur, n_pi, n_pir,
             n_nu, n_nur, n_ni, n_nir,
             *scratch):
    raw_v = scratch[0:8]
    full_v = scratch[8:20]
    srow_v = scratch[20:32]
    rows_v = scratch[32:44]
    acc_v = scratch[44:57]
    sem = scratch[57]

    wid = lax.axis_index("s") * _NC + lax.axis_index("c")
    base = wid * _ROWS_PER_W

    raw_in = [user_idx, item_idx, pos_user_idx, pos_item_idx,
              neg_user_idx, neg_item_idx, rel_idx, neg_rel_idx]
    idx_descs = [
        pltpu.async_copy(src.at[pl.ds(base, _ROWS_PER_W)], dst, sem)
        for src, dst in zip(raw_in, raw_v)
    ]
    for d in idx_descs:
        d.wait()

    def _derive(t, carry):
        s = pl.ds(t * 16, 16)
        u, it, pu, pi_, nu, ni, r, nr = (rv[s] for rv in raw_v)
        full = [u, u + r * _USER_NUM, it, it + r * _ITEM_NUM,
                pu, pu + r * _USER_NUM, pi_, pi_ + r * _ITEM_NUM,
                nu, nu + nr * _USER_NUM, ni, ni + nr * _ITEM_NUM]
        for m in range(12):
            full_v[m][s] = full[m]
            srow_v[m][s] = lax.shift_right_logical(full[m], 2)
        return carry
    lax.fori_loop(0, _ROWS_PER_W // 16, _derive, 0)

    tables = [user_t, urel_t, item_t, irel_t,
              user_t, urel_t, item_t, irel_t,
              user_t, urel_t, item_t, irel_t]

    lane = lax.iota(jnp.int32, 16)

    def _chunk(j, carry):
        off = j * _CH
        descs = [
            pltpu.async_copy(tables[m].at[srow_v[m].at[pl.ds(off, _CH)]],
                             rows_v[m], sem)
            for m in range(12)
        ]
        for d in descs:
            d.wait()

        def _block(b, carry2):
            boff = b * 16
            rows = boff + lane
            colbase = [
                lax.bitwise_and(full_v[m][pl.ds(off + boff, 16)], 3) * _FACTOR
                for m in range(12)
            ]

            def _col(c, acc):
                v = [plsc.load_gather(rows_v[m], [rows, colbase[m] + c])
                     for m in range(12)]
                xa = acc[0] + ((v[0] + v[1]) + (v[2] + v[3])) * (
                    ((v[4] + v[5]) + (v[6] + v[7]))
                    - ((v[8] + v[9]) + (v[10] + v[11])))
                ns = tuple(acc[1 + m] + v[m] * v[m] for m in range(12))
                return (xa,) + ns

            z = jnp.zeros((16,), jnp.float32)
            acc = plsc.parallel_loop(0, _FACTOR, unroll=4, carry=(z,) * 13)(
                _col)
            for m in range(13):
                acc_v[m][pl.ds(off + boff, 16)] = acc[m]
            return carry2
        lax.fori_loop(0, _NBLK, _block, 0)
        return carry
    lax.fori_loop(0, _NCHUNK, _chunk, 0)

    out_by_acc = [xhat_out, n_u, n_ur, n_i, n_ir, n_pu, n_pur, n_pi,
                  n_pir, n_nu, n_nur, n_ni, n_nir]
    for a, o in zip(acc_v, out_by_acc):
        pltpu.sync_copy(a, o.at[pl.ds(base, _ROWS_PER_W)])


def _finish_body(x_ref, *rest):
    n_refs = rest[:12]
    loss_ref, reg_ref = rest[12], rest[13]
    x = x_ref[...]
    loss_ref[0, 0] = jnp.sum(jnp.log(1.0 + jnp.exp(-x)))
    acc = jnp.zeros((), jnp.float32)
    for r in n_refs:
        acc = acc + jnp.sum(jnp.sqrt(r[...]))
    reg_ref[0, 0] = acc * _LAMDA


def kernel(user_idx, item_idx, pos_user_idx, pos_item_idx, neg_user_idx,
           neg_item_idx, rel_idx, neg_rel_idx, user_table, item_table,
           urel_table, irel_table):
    mesh = plsc.VectorSubcoreMesh(core_axis_name="c", subcore_axis_name="s")
    out13 = [jax.ShapeDtypeStruct((_BATCH,), jnp.float32)] * 13
    scratch = (
        [pltpu.VMEM((_ROWS_PER_W,), jnp.int32)] * 8
        + [pltpu.VMEM((_ROWS_PER_W,), jnp.int32)] * 12
        + [pltpu.VMEM((_ROWS_PER_W,), jnp.int32)] * 12
        + [pltpu.VMEM((_CH, 128), jnp.float32)] * 12
        + [pltpu.VMEM((_ROWS_PER_W,), jnp.float32)] * 13
        + [pltpu.SemaphoreType.DMA]
    )
    sc = pl.kernel(
        _sc_body,
        out_type=out13,
        mesh=mesh,
        scratch_types=scratch,
        compiler_params=pltpu.CompilerParams(needs_layout_passes=False),
    )
    parts = sc(user_idx.astype(jnp.int32), item_idx.astype(jnp.int32),
               pos_user_idx.astype(jnp.int32), pos_item_idx.astype(jnp.int32),
               neg_user_idx.astype(jnp.int32), neg_item_idx.astype(jnp.int32),
               rel_idx.astype(jnp.int32), neg_rel_idx.astype(jnp.int32),
               user_table.reshape(_USER_NUM // 4, 128),
               item_table.reshape(_ITEM_NUM // 4, 128),
               urel_table.reshape(_USER_NUM * 3 // 4, 128),
               irel_table.reshape(_ITEM_NUM * 3 // 4, 128))

    shaped = [p.reshape(128, 128) for p in parts]
    loss, reg = pl.pallas_call(
        _finish_body,
        out_shape=[jax.ShapeDtypeStruct((1, 1), jnp.float32)] * 2,
        out_specs=[pl.BlockSpec(memory_space=pltpu.MemorySpace.SMEM)] * 2,
    )(*shaped)
    return (loss[0, 0], reg[0, 0])


# speedup vs baseline: 1.3229x; 1.3229x over previous
import jax
import jax.numpy as jnp
from jax import lax
from jax.experimental import pallas as pl
from jax.experimental.pallas import tpu as pltpu
from jax.experimental.pallas import tpu_sc as plsc

_USER_NUM = 1000000
_ITEM_NUM = 100000
_FACTOR = 32
_BATCH = 16384
_LAMDA = 0.001

_NC = 2
_NS = 16
_NW = _NC * _NS
_ROWS_PER_W = _BATCH // _NW
_CH = 64
_NCHUNK = _ROWS_PER_W // _CH
_NBLK = _CH // 16


def _sc_body(user_idx, item_idx, pos_user_idx, pos_item_idx, neg_user_idx,
             neg_item_idx, rel_idx, neg_rel_idx,
             user_table, item_table, urel_table, irel_table,
             xhat_out, n_u, n_ur, n_i, n_ir, n_pu, n_pur, n_pi, n_pir,
             n_nu, n_nur, n_ni, n_nir,
             *scratch):
    raw_v = scratch[0:8]
    drv_v = scratch[8:14]
    rows_v = scratch[14:26]
    acc_v = scratch[26:39]
    sem = scratch[39]

    wid = lax.axis_index("s") * _NC + lax.axis_index("c")
    base = wid * _ROWS_PER_W

    raw_in = [user_idx, item_idx, pos_user_idx, pos_item_idx,
              neg_user_idx, neg_item_idx, rel_idx, neg_rel_idx]
    for src, dst in zip(raw_in, raw_v):
        pltpu.sync_copy(src.at[pl.ds(base, _ROWS_PER_W)], dst)

    def _derive(t, carry):
        s = pl.ds(t * 16, 16)
        r = raw_v[6][s]
        nr = raw_v[7][s]
        drv_v[0][s] = raw_v[0][s] + r * _USER_NUM
        drv_v[1][s] = raw_v[2][s] + r * _USER_NUM
        drv_v[2][s] = raw_v[4][s] + nr * _USER_NUM
        drv_v[3][s] = raw_v[1][s] + r * _ITEM_NUM
        drv_v[4][s] = raw_v[3][s] + r * _ITEM_NUM
        drv_v[5][s] = raw_v[5][s] + nr * _ITEM_NUM
        return carry
    lax.fori_loop(0, _ROWS_PER_W // 16, _derive, 0)

    gathers = [
        (user_table, raw_v[0]), (urel_table, drv_v[0]),
        (item_table, raw_v[1]), (irel_table, drv_v[3]),
        (user_table, raw_v[2]), (urel_table, drv_v[1]),
        (item_table, raw_v[3]), (irel_table, drv_v[4]),
        (user_table, raw_v[4]), (urel_table, drv_v[2]),
        (item_table, raw_v[5]), (irel_table, drv_v[5]),
    ]

    lane = lax.iota(jnp.int32, 16)

    for j in range(_NCHUNK):
        def _issue(b, carry):
            for m, (tbl, iv) in enumerate(gathers):
                vec = iv[pl.ds(j * _CH + b * 16, 16)]
                for k in range(16):
                    pltpu.async_copy(
                        tbl.at[pl.ds(vec[k], 1)],
                        rows_v[m].at[pl.ds(b * 16 + k, 1)],
                        sem)
            return carry
        lax.fori_loop(0, _NBLK, _issue, 0)
        for m in range(12):
            pltpu.make_async_copy(user_table.at[pl.ds(0, _CH)], rows_v[m],
                                  sem).wait()

        def _block(b, carry):
            rows = b * 16 + lane

            def _col(c, acc):
                cols = jnp.full((16,), c, jnp.int32)
                v = [plsc.load_gather(rows_v[m], [rows, cols])
                     for m in range(12)]
                xa = acc[0] + ((v[0] + v[1]) + (v[2] + v[3])) * (
                    ((v[4] + v[5]) + (v[6] + v[7]))
                    - ((v[8] + v[9]) + (v[10] + v[11])))
                ns = tuple(acc[1 + m] + v[m] * v[m] for m in range(12))
                return (xa,) + ns

            z = jnp.zeros((16,), jnp.float32)
            acc = lax.fori_loop(0, _FACTOR, _col, (z,) * 13)
            off = j * _CH + b * 16
            for m in range(13):
                acc_v[m][pl.ds(off, 16)] = acc[m]
            return carry
        lax.fori_loop(0, _NBLK, _block, 0)

    out_by_acc = [xhat_out, n_u, n_ur, n_i, n_ir, n_pu, n_pur, n_pi,
                  n_pir, n_nu, n_nur, n_ni, n_nir]
    for a, o in zip(acc_v, out_by_acc):
        pltpu.sync_copy(a, o.at[pl.ds(base, _ROWS_PER_W)])


def _finish_body(x_ref, *rest):
    n_refs = rest[:12]
    loss_ref, reg_ref = rest[12], rest[13]
    x = x_ref[...]
    loss_ref[0, 0] = jnp.sum(jnp.log(1.0 + jnp.exp(-x)))
    acc = jnp.zeros((), jnp.float32)
    for r in n_refs:
        acc = acc + jnp.sum(jnp.sqrt(r[...]))
    reg_ref[0, 0] = acc * _LAMDA


def kernel(user_idx, item_idx, pos_user_idx, pos_item_idx, neg_user_idx,
           neg_item_idx, rel_idx, neg_rel_idx, user_table, item_table,
           urel_table, irel_table):
    mesh = plsc.VectorSubcoreMesh(core_axis_name="c", subcore_axis_name="s")
    out13 = [jax.ShapeDtypeStruct((_BATCH,), jnp.float32)] * 13
    scratch = (
        [pltpu.VMEM((_ROWS_PER_W,), jnp.int32)] * 8
        + [pltpu.VMEM((_ROWS_PER_W,), jnp.int32)] * 6
        + [pltpu.VMEM((_CH, _FACTOR), jnp.float32)] * 12
        + [pltpu.VMEM((_ROWS_PER_W,), jnp.float32)] * 13
        + [pltpu.SemaphoreType.DMA]
    )
    sc = pl.kernel(
        _sc_body,
        out_type=out13,
        mesh=mesh,
        scratch_types=scratch,
        compiler_params=pltpu.CompilerParams(
            needs_layout_passes=False, use_tc_tiling_on_sc=True),
    )
    parts = sc(user_idx.astype(jnp.int32), item_idx.astype(jnp.int32),
               pos_user_idx.astype(jnp.int32), pos_item_idx.astype(jnp.int32),
               neg_user_idx.astype(jnp.int32), neg_item_idx.astype(jnp.int32),
               rel_idx.astype(jnp.int32), neg_rel_idx.astype(jnp.int32),
               user_table, item_table, urel_table, irel_table)

    shaped = [p.reshape(128, 128) for p in parts]
    loss, reg = pl.pallas_call(
        _finish_body,
        out_shape=[jax.ShapeDtypeStruct((1, 1), jnp.float32)] * 2,
        out_specs=[pl.BlockSpec(memory_space=pltpu.MemorySpace.SMEM)] * 2,
    )(*shaped)
    return (loss[0, 0], reg[0, 0])
